# baseline (speedup 1.0000x reference)
import functools

import jax
import jax.numpy as jnp
from jax import lax
from jax.experimental import pallas as pl
from jax.experimental.pallas import tpu as pltpu
from jax.experimental.pallas import tpu_sc as plsc

VOCAB = 1000000
EMBED_DIM = 64
NUM_CORES = 2
NUM_SUBCORES = 16
NUM_WORKERS = NUM_CORES * NUM_SUBCORES

N_FULL_BLOCKS = VOCAB // 128
RAGGED_COLS = VOCAB - N_FULL_BLOCKS * 128
BLOCKS_PER_W = -(-N_FULL_BLOCKS // NUM_WORKERS)
LAST_COMMON_I = 243


def _transpose_block(in_buf, out_buf, iotas, n_rows):
  def row(r, carry):
    for ci, c0 in enumerate(range(0, 128, 16)):
      half = c0 // 64
      idx_c = iotas[ci] * 0 + (2 * r + half)
      vals = plsc.load_gather(in_buf, [iotas[ci], idx_c])
      out_buf[r, pl.ds(c0, 16)] = vals
    return carry

  lax.fori_loop(0, n_rows, row, 0)


@functools.lru_cache(maxsize=None)
def _make_transpose():
  mesh = plsc.VectorSubcoreMesh(core_axis_name="c", subcore_axis_name="s")

  @functools.partial(
      pl.kernel,
      mesh=mesh,
      compiler_params=pltpu.CompilerParams(
          use_tc_tiling_on_sc=True, needs_layout_passes=False),
      out_type=jax.ShapeDtypeStruct((VOCAB // 2, 128), jnp.float32),
      scratch_types=[
          pltpu.VMEM((64, 128), jnp.float32),
          pltpu.VMEM((64, 128), jnp.float32),
          pltpu.VMEM((64, 128), jnp.float32),
          pltpu.VMEM((64, 128), jnp.float32),
          pltpu.SemaphoreType.DMA,
          pltpu.SemaphoreType.DMA,
          pltpu.SemaphoreType.DMA,
          pltpu.SemaphoreType.DMA,
      ],
  )
  def transpose_kernel(tt_hbm, last2_hbm, out_hbm, in0, in1, out0, out1,
                       sem_i0, sem_i1, sem_o0, sem_o1):
    wid = lax.axis_index("s") * NUM_CORES + lax.axis_index("c")
    iota16 = lax.iota(jnp.int32, 16)
    iotas = [iota16 + (c0 % 64) for c0 in range(0, 128, 16)]

    def blk(i):
      return wid + NUM_WORKERS * i

    def start_in(buf, sem, i):
      pltpu.async_copy(tt_hbm.at[:, pl.ds(blk(i) * 128, 128)], buf, sem)

    def wait_in(buf, sem, i):
      pltpu.make_async_copy(
          tt_hbm.at[:, pl.ds(blk(i) * 128, 128)], buf, sem).wait()

    def start_out(buf, sem, i):
      pltpu.async_copy(buf, out_hbm.at[pl.ds(blk(i) * 64, 64)], sem)

    def wait_out(buf, sem, i):
      pltpu.make_async_copy(
          buf, out_hbm.at[pl.ds(blk(i) * 64, 64)], sem).wait()

    start_in(in0, sem_i0, 0)
    start_in(in1, sem_i1, 1)

    wait_in(in0, sem_i0, 0)
    _transpose_block(in0, out0, iotas, 64)
    start_out(out0, sem_o0, 0)
    start_in(in0, sem_i0, 2)
    wait_in(in1, sem_i1, 1)
    _transpose_block(in1, out1, iotas, 64)
    start_out(out1, sem_o1, 1)
    start_in(in1, sem_i1, 3)

    def body(g, carry):
      i0 = 2 * g
      i1 = i0 + 1
      wait_in(in0, sem_i0, i0)
      wait_out(out0, sem_o0, i0 - 2)
      _transpose_block(in0, out0, iotas, 64)
      start_out(out0, sem_o0, i0)
      start_in(in0, sem_i0, i0 + 2)
      wait_in(in1, sem_i1, i1)
      wait_out(out1, sem_o1, i1 - 2)
      _transpose_block(in1, out1, iotas, 64)
      start_out(out1, sem_o1, i1)
      start_in(in1, sem_i1, i1 + 2)
      return carry

    lax.fori_loop(1, 121, body, 0)

    wait_in(in0, sem_i0, 242)
    wait_out(out0, sem_o0, 240)
    _transpose_block(in0, out0, iotas, 64)
    start_out(out0, sem_o0, 242)

    @pl.when(blk(244) < N_FULL_BLOCKS)
    def _():
      start_in(in0, sem_i0, 244)

    wait_in(in1, sem_i1, 243)
    wait_out(out1, sem_o1, 241)
    _transpose_block(in1, out1, iotas, 64)
    start_out(out1, sem_o1, 243)
    wait_out(out0, sem_o0, 242)

    @pl.when(blk(244) < N_FULL_BLOCKS)
    def _():
      wait_in(in0, sem_i0, 244)
      _transpose_block(in0, out0, iotas, 64)
      start_out(out0, sem_o0, 244)
      wait_out(out0, sem_o0, 244)

    wait_out(out1, sem_o1, 243)

    @pl.when(wid == 4)
    def _():
      pltpu.sync_copy(last2_hbm, in0.at[pl.ds(0, 32)])
      pltpu.sync_copy(in0.at[pl.ds(0, 32)],
                      out_hbm.at[pl.ds(N_FULL_BLOCKS * 64, 32)])

  return transpose_kernel


@functools.lru_cache(maxsize=None)
def _make_gather(B, chunk):
  b_per_w = B // NUM_WORKERS
  n_chunks = b_per_w // chunk
  assert b_per_w % chunk == 0 and n_chunks % 2 == 0
  n2 = n_chunks // 2
  mesh = plsc.VectorSubcoreMesh(core_axis_name="c", subcore_axis_name="s")

  @functools.partial(
      pl.kernel,
      mesh=mesh,
      compiler_params=pltpu.CompilerParams(use_tc_tiling_on_sc=False),
      out_type=jax.ShapeDtypeStruct((B, EMBED_DIM), jnp.float32),
      scratch_types=[
          pltpu.VMEM((b_per_w,), jnp.int32),
          pltpu.VMEM((chunk, EMBED_DIM), jnp.float32),
          pltpu.VMEM((chunk, EMBED_DIM), jnp.float32),
          pltpu.SemaphoreType.DMA,
          pltpu.SemaphoreType.DMA,
          pltpu.SemaphoreType.DMA,
          pltpu.SemaphoreType.DMA,
      ],
  )
  def gather_kernel(table_hbm, idx_hbm, out_hbm, idx_v, rows0, rows1,
                    sem_g0, sem_g1, sem_o0, sem_o1):
    wid = lax.axis_index("s") * NUM_CORES + lax.axis_index("c")
    base = wid * b_per_w
    pltpu.sync_copy(idx_hbm.at[pl.ds(base, b_per_w)], idx_v)

    def idx_slice(c):
      return idx_v.at[pl.ds(c * chunk, chunk)]

    def start_gather(rows, sem, c):
      pltpu.async_copy(table_hbm.at[idx_slice(c)], rows, sem)

    def wait_gather(rows, sem, c):
      pltpu.make_async_copy(table_hbm.at[idx_slice(c)], rows, sem).wait()

    def start_out(rows, sem, c):
      pltpu.async_copy(rows, out_hbm.at[pl.ds(base + c * chunk, chunk)], sem)

    def wait_out(rows, sem, c):
      pltpu.make_async_copy(
          rows, out_hbm.at[pl.ds(base + c * chunk, chunk)], sem).wait()

    start_gather(rows0, sem_g0, 0)
    wait_gather(rows0, sem_g0, 0)
    start_gather(rows1, sem_g1, 1)
    start_out(rows0, sem_o0, 0)
    wait_gather(rows1, sem_g1, 1)
    wait_out(rows0, sem_o0, 0)
    start_gather(rows0, sem_g0, 2)
    start_out(rows1, sem_o1, 1)

    def body(g2, carry):
      c0 = 2 * g2
      c1 = c0 + 1
      c2 = c0 + 2
      wait_gather(rows0, sem_g0, c0)
      wait_out(rows1, sem_o1, c1 - 2)
      start_gather(rows1, sem_g1, c1)
      start_out(rows0, sem_o0, c0)
      wait_gather(rows1, sem_g1, c1)
      wait_out(rows0, sem_o0, c0)
      start_gather(rows0, sem_g0, c2)
      start_out(rows1, sem_o1, c1)
      return carry

    lax.fori_loop(1, n2 - 1, body, 0)

    c0 = n_chunks - 2
    c1 = n_chunks - 1
    wait_gather(rows0, sem_g0, c0)
    wait_out(rows1, sem_o1, c1 - 2)
    start_gather(rows1, sem_g1, c1)
    start_out(rows0, sem_o0, c0)
    wait_gather(rows1, sem_g1, c1)
    wait_out(rows0, sem_o0, c0)
    start_out(rows1, sem_o1, c1)
    wait_out(rows1, sem_o1, c1)

  return gather_kernel


def kernel(input_x, table):
  batch, hist = input_x.shape
  B = batch * hist
  tt = jnp.swapaxes(table, 0, 1)
  last2 = table[N_FULL_BLOCKS * 128:].reshape(32, 128)
  lin = _make_transpose()(tt, last2)
  tbl = lin.reshape(VOCAB, EMBED_DIM)
  idx = input_x.reshape(B).astype(jnp.int32)
  out = _make_gather(B, 512)(tbl, idx)
  emb = out.reshape(batch, hist, EMBED_DIM)
  return (emb, emb)


# speedup vs baseline: 1.6435x; 1.6435x over previous
import functools

import jax
import jax.numpy as jnp
from jax import lax
from jax.experimental import pallas as pl
from jax.experimental.pallas import tpu as pltpu
from jax.experimental.pallas import tpu_sc as plsc

EMBED_DIM = 64
NUM_CORES = 2
NUM_SUBCORES = 16
NUM_WORKERS = NUM_CORES * NUM_SUBCORES


@functools.lru_cache(maxsize=None)
def _make_gather(B, chunk):
  b_per_w = B // NUM_WORKERS
  n_chunks = b_per_w // chunk
  assert b_per_w % chunk == 0 and n_chunks % 2 == 0
  n2 = n_chunks // 2
  mesh = plsc.VectorSubcoreMesh(core_axis_name="c", subcore_axis_name="s")

  @functools.partial(
      pl.kernel,
      mesh=mesh,
      compiler_params=pltpu.CompilerParams(use_tc_tiling_on_sc=False),
      out_type=jax.ShapeDtypeStruct((B, EMBED_DIM), jnp.float32),
      scratch_types=[
          pltpu.VMEM((b_per_w,), jnp.int32),
          pltpu.VMEM((chunk, EMBED_DIM), jnp.float32),
          pltpu.VMEM((chunk, EMBED_DIM), jnp.float32),
          pltpu.SemaphoreType.DMA,
          pltpu.SemaphoreType.DMA,
          pltpu.SemaphoreType.DMA,
          pltpu.SemaphoreType.DMA,
      ],
  )
  def gather_kernel(table_hbm, idx_hbm, out_hbm, idx_v, rows0, rows1,
                    sem_g0, sem_g1, sem_o0, sem_o1):
    wid = lax.axis_index("s") * NUM_CORES + lax.axis_index("c")
    base = wid * b_per_w
    pltpu.sync_copy(idx_hbm.at[pl.ds(base, b_per_w)], idx_v)

    def idx_slice(c):
      return idx_v.at[pl.ds(c * chunk, chunk)]

    def start_gather(rows, sem, c):
      pltpu.async_copy(table_hbm.at[idx_slice(c)], rows, sem)

    def wait_gather(rows, sem, c):
      pltpu.make_async_copy(table_hbm.at[idx_slice(c)], rows, sem).wait()

    def start_out(rows, sem, c):
      pltpu.async_copy(rows, out_hbm.at[pl.ds(base + c * chunk, chunk)], sem)

    def wait_out(rows, sem, c):
      pltpu.make_async_copy(
          rows, out_hbm.at[pl.ds(base + c * chunk, chunk)], sem).wait()

    start_gather(rows0, sem_g0, 0)
    wait_gather(rows0, sem_g0, 0)
    start_gather(rows1, sem_g1, 1)
    start_out(rows0, sem_o0, 0)
    wait_gather(rows1, sem_g1, 1)
    wait_out(rows0, sem_o0, 0)
    start_gather(rows0, sem_g0, 2)
    start_out(rows1, sem_o1, 1)

    def body(g2, carry):
      c0 = 2 * g2
      c1 = c0 + 1
      c2 = c0 + 2
      wait_gather(rows0, sem_g0, c0)
      wait_out(rows1, sem_o1, c1 - 2)
      start_gather(rows1, sem_g1, c1)
      start_out(rows0, sem_o0, c0)
      wait_gather(rows1, sem_g1, c1)
      wait_out(rows0, sem_o0, c0)
      start_gather(rows0, sem_g0, c2)
      start_out(rows1, sem_o1, c1)
      return carry

    lax.fori_loop(1, n2 - 1, body, 0)

    c0 = n_chunks - 2
    c1 = n_chunks - 1
    wait_gather(rows0, sem_g0, c0)
    wait_out(rows1, sem_o1, c1 - 2)
    start_gather(rows1, sem_g1, c1)
    start_out(rows0, sem_o0, c0)
    wait_gather(rows1, sem_g1, c1)
    wait_out(rows0, sem_o0, c0)
    start_out(rows1, sem_o1, c1)
    wait_out(rows1, sem_o1, c1)

  return gather_kernel


def kernel(input_x, table):
  batch, hist = input_x.shape
  B = batch * hist
  idx = input_x.reshape(B).astype(jnp.int32)
  out = _make_gather(B, 512)(table, idx)
  emb = out.reshape(batch, hist, EMBED_DIM)
  return (emb, emb)
